# baseline (speedup 1.0000x reference)
import functools

import jax
import jax.numpy as jnp
from jax import lax
from jax.experimental import pallas as pl
from jax.experimental.pallas import tpu as pltpu
from jax.experimental.pallas import tpu_sc as plsc

NC = 2
NS = 16
NW = NC * NS
BB = 128


def _body(L, D_char, srcT_r, fT_r, char_r, ftab_r, out_r,
          sidx_v, fidx_v, ftab_v, char_v, asm_v, g0, g1, w0, w1):
    F = ftab_v.shape[1]
    DG_C = D_char // 8
    DG = asm_v.shape[1]
    gsem = (g0, g1)
    wsem = (w0, w1)
    w = lax.axis_index("s") * NC + lax.axis_index("c")
    wb = w * BB

    pltpu.sync_copy(srcT_r.at[:, pl.ds(wb, BB)], sidx_v)
    pltpu.sync_copy(fT_r.at[:, :, pl.ds(wb, BB)], fidx_v)
    pltpu.sync_copy(ftab_r, ftab_v)

    iota = lax.iota(jnp.int32, 16)
    rowv = [bg * 16 + iota for bg in range(BB // 16)]

    def fire_gather(l, p):
        pltpu.async_copy(char_r.at[sidx_v.at[l, :]], char_v.at[p], gsem[p])

    def wait_gather(p):
        pltpu.make_async_copy(char_r.at[pl.ds(0, BB), :],
                              char_v.at[p], gsem[p]).wait()

    def fire_write(l, p):
        pltpu.async_copy(asm_v.at[p], out_r.at[l, :, w], wsem[p])

    def drain_write(p):
        pltpu.make_async_copy(asm_v.at[p], out_r.at[0, :, 0], wsem[p]).wait()

    fire_gather(0, 0)

    def outer(l2, carry):
        for p in range(2):
            l = l2 * 2 + p

            @pl.when(l + 1 < L)
            def _():
                fire_gather(l + 1, 1 - p)

            wait_gather(p)

            @pl.when(l >= 2)
            def _():
                drain_write(p)

            cb = char_v.at[p]
            ab = asm_v.at[p]

            def char_tile(dg, c):
                for dr in range(8):
                    dv = jnp.full((16,), dg * 8 + dr, jnp.int32)
                    for bg in range(BB // 16):
                        v = plsc.load_gather(cb, [rowv[bg], dv])
                        ab[dg, dr, pl.ds(bg * 16, 16)] = v
                return c

            lax.fori_loop(0, DG_C, char_tile, 0)

            def feat_tile(dg, c):
                for dr in range(8):
                    e = dg * 8 + dr - D_char
                    i = e // 16
                    kv = jnp.full((16,), e % 16, jnp.int32)
                    iv = jnp.full((16,), i, jnp.int32)
                    for bg in range(BB // 16):
                        fi = fidx_v[l, i, pl.ds(bg * 16, 16)]
                        v = plsc.load_gather(ftab_v, [kv, iv, fi])
                        ab[dg, dr, pl.ds(bg * 16, 16)] = v
                return c

            lax.fori_loop(DG_C, DG, feat_tile, 0)

            fire_write(l, p)
        return carry

    lax.fori_loop(0, L // 2, outer, 0)
    drain_write(0)
    drain_write(1)


def kernel(src, feats, char_table, feat_tables):
    B, L = src.shape
    assert B == NW * BB and L % 2 == 0
    F, V_f, D_feat = feat_tables.shape
    D_char = char_table.shape[1]
    D = D_char + F * D_feat
    assert D % 8 == 0 and D_char % 8 == 0 and BB % 128 == 0

    srcT = src.astype(jnp.int32).T
    fT = feats.astype(jnp.int32).transpose(2, 0, 1)
    ftabT = feat_tables.transpose(2, 0, 1)

    mesh = plsc.VectorSubcoreMesh(
        core_axis_name="c", subcore_axis_name="s",
        num_cores=NC, num_subcores=NS)

    run = pl.kernel(
        functools.partial(_body, L, D_char),
        out_type=jax.ShapeDtypeStruct((L, D // 8, NW, 8, BB), jnp.float32),
        mesh=mesh,
        scratch_types=[
            pltpu.VMEM((L, BB), jnp.int32),
            pltpu.VMEM((L, F, BB), jnp.int32),
            pltpu.VMEM((D_feat, F, V_f), jnp.float32),
            pltpu.VMEM((2, BB, D_char), jnp.float32),
            pltpu.VMEM((2, D // 8, 8, BB), jnp.float32),
        ] + [pltpu.SemaphoreType.DMA] * 4,
        compiler_params=pltpu.CompilerParams(use_tc_tiling_on_sc=False,
                                             needs_layout_passes=False),
    )
    out5 = run(srcT, fT, char_table, ftabT)
    return out5.transpose(2, 4, 0, 1, 3).reshape(B, L, D)


# speedup vs baseline: 2.2083x; 2.2083x over previous
import functools

import jax
import jax.numpy as jnp
from jax import lax
from jax.experimental import pallas as pl
from jax.experimental.pallas import tpu as pltpu
from jax.experimental.pallas import tpu_sc as plsc

NC = 2
NS = 16
NW = NC * NS
BB = 128


def _body(L, D_char, srcT_r, fT_r, char_r, ftab_r, out_r,
          sidx_v, fidx_v, ftab_v, char_v, asm_v, g0, g1, w0, w1):
    F = ftab_v.shape[1]
    DG_C = D_char // 8
    DG = asm_v.shape[1]
    gsem = (g0, g1)
    wsem = (w0, w1)
    w = lax.axis_index("s") * NC + lax.axis_index("c")
    wb = w * BB

    pltpu.sync_copy(srcT_r.at[:, pl.ds(wb, BB)], sidx_v)
    pltpu.sync_copy(fT_r.at[:, :, pl.ds(wb, BB)], fidx_v)
    pltpu.sync_copy(ftab_r, ftab_v)

    iota = lax.iota(jnp.int32, 16)
    rowv = [bg * 16 + iota for bg in range(BB // 16)]

    def fire_gather(l, p):
        pltpu.async_copy(char_r.at[sidx_v.at[l, :]], char_v.at[p], gsem[p])

    def wait_gather(p):
        pltpu.make_async_copy(char_r.at[pl.ds(0, BB), :],
                              char_v.at[p], gsem[p]).wait()

    def fire_write(l, p):
        pltpu.async_copy(asm_v.at[p], out_r.at[l, :, w], wsem[p])

    def drain_write(p):
        pltpu.make_async_copy(asm_v.at[p], out_r.at[0, :, 0], wsem[p]).wait()

    fire_gather(0, 0)

    def outer(l2, carry):
        for p in range(2):
            l = l2 * 2 + p

            @pl.when(l + 1 < L)
            def _():
                fire_gather(l + 1, 1 - p)

            wait_gather(p)

            @pl.when(l >= 2)
            def _():
                drain_write(p)

            cb = char_v.at[p]
            ab = asm_v.at[p]

            @plsc.parallel_loop(0, D_char, unroll=2)
            def _(d):
                dv = jnp.full((16,), d, jnp.int32)
                dg = d // 8
                dr = d % 8
                for bg in range(BB // 16):
                    v = plsc.load_gather(cb, [rowv[bg], dv])
                    ab[dg, dr, pl.ds(bg * 16, 16)] = v

            @plsc.parallel_loop(0, DG * 8 - D_char, unroll=2)
            def _(e):
                i = e // 16
                kv = jnp.full((16,), e % 16, jnp.int32)
                iv = jnp.full((16,), i, jnp.int32)
                d = D_char + e
                dg = d // 8
                dr = d % 8
                for bg in range(BB // 16):
                    fi = fidx_v[l, i, pl.ds(bg * 16, 16)]
                    v = plsc.load_gather(ftab_v, [kv, iv, fi])
                    ab[dg, dr, pl.ds(bg * 16, 16)] = v

            fire_write(l, p)
        return carry

    lax.fori_loop(0, L // 2, outer, 0)
    drain_write(0)
    drain_write(1)


def kernel(src, feats, char_table, feat_tables):
    B, L = src.shape
    assert B == NW * BB and L % 2 == 0
    F, V_f, D_feat = feat_tables.shape
    D_char = char_table.shape[1]
    D = D_char + F * D_feat
    assert D % 8 == 0 and D_char % 8 == 0 and BB % 128 == 0

    srcT = src.astype(jnp.int32).T
    fT = feats.astype(jnp.int32).transpose(2, 0, 1)
    ftabT = feat_tables.transpose(2, 0, 1)

    mesh = plsc.VectorSubcoreMesh(
        core_axis_name="c", subcore_axis_name="s",
        num_cores=NC, num_subcores=NS)

    run = pl.kernel(
        functools.partial(_body, L, D_char),
        out_type=jax.ShapeDtypeStruct((L, D // 8, NW, 8, BB), jnp.float32),
        mesh=mesh,
        scratch_types=[
            pltpu.VMEM((L, BB), jnp.int32),
            pltpu.VMEM((L, F, BB), jnp.int32),
            pltpu.VMEM((D_feat, F, V_f), jnp.float32),
            pltpu.VMEM((2, BB, D_char), jnp.float32),
            pltpu.VMEM((2, D // 8, 8, BB), jnp.float32),
        ] + [pltpu.SemaphoreType.DMA] * 4,
        compiler_params=pltpu.CompilerParams(use_tc_tiling_on_sc=False,
                                             needs_layout_passes=False),
    )
    out5 = run(srcT, fT, char_table, ftabT)
    return out5.transpose(2, 4, 0, 1, 3).reshape(B, L, D)
